# baseline (speedup 1.0000x reference)
import functools

import jax
import jax.numpy as jnp
from jax import lax
from jax.experimental import pallas as pl
from jax.experimental.pallas import tpu as pltpu
from jax.experimental.pallas import tpu_sc as plsc

_L = 16


@functools.lru_cache(maxsize=None)
def _build(Br, F, V, D, NC, NS):
    NW = NC * NS
    bpw = Br // NW
    rpw = bpw * F
    NG = bpw // _L
    mesh = plsc.VectorSubcoreMesh(core_axis_name="c", subcore_axis_name="s")

    @functools.partial(
        pl.kernel,
        mesh=mesh,
        compiler_params=pltpu.CompilerParams(
            use_tc_tiling_on_sc=True, needs_layout_passes=False
        ),
        out_type=jax.ShapeDtypeStruct((F, D, Br), jnp.float32),
        scratch_types=[
            pltpu.VMEM((rpw,), jnp.int32),
            pltpu.VMEM((bpw,), jnp.int32),
            pltpu.VMEM((bpw, 2 * D), jnp.float32),
            pltpu.VMEM((D, bpw), jnp.float32),
            pltpu.SemaphoreType.DMA,
        ],
    )
    def emb(idx_hbm, table_hbm, out_hbm, idx_v, pid_v, buf, obuf, gsem):
        wid = lax.axis_index("s") * NC + lax.axis_index("c")
        pltpu.sync_copy(idx_hbm.at[pl.ds(wid * rpw, rpw)], idx_v)

        lane = lax.iota(jnp.int32, _L)
        posg = [lane * F + g * _L * F for g in range(NG)]
        rowg = [lane + g * _L for g in range(NG)]

        def per_field(f, carry):
            for g in range(NG):
                raw = plsc.load_gather(idx_v, [posg[g] + f])
                pid_v[pl.ds(g * _L, _L)] = raw
            pltpu.async_copy(table_hbm.at[pid_v], buf, gsem).wait()

            def per_dim(d, c2):
                for g in range(NG):
                    vals = plsc.load_gather(buf, [rowg[g], jnp.full((_L,), 0, jnp.int32) + d])
                    obuf[d, pl.ds(g * _L, _L)] = vals
                return c2

            lax.fori_loop(0, D, per_dim, 0)
            pltpu.sync_copy(obuf, out_hbm.at[f, :, pl.ds(wid * bpw, bpw)])
            return carry

        lax.fori_loop(0, F, per_field, 0)

    return emb


def kernel(input, emb_weight):
    Br, F = input.shape
    V, D = emb_weight.shape
    info = plsc.get_sparse_core_info()
    NC, NS = info.num_cores, info.num_subcores
    idx = input.reshape(-1).astype(jnp.int32)
    table2 = jnp.pad(emb_weight, ((0, 0), (0, 2 * D - D)))
    out_t = _build(Br, F, V, D, NC, NS)(idx, table2)
    return jnp.transpose(out_t, (2, 0, 1))
